# baseline (speedup 1.0000x reference)
import functools

import jax
import jax.numpy as jnp
from jax import lax
from jax.experimental import pallas as pl
from jax.experimental.pallas import tpu as pltpu
from jax.experimental.pallas import tpu_sc as plsc

B = 16
N = 2048
M = 2048
LANES = 16
NUM_CORES = 2
NUM_SUBCORES = 16
NUM_WORKERS = NUM_CORES * NUM_SUBCORES
G = 8

SC_BATCHES = 2
TC_BATCHES = B - SC_BATCHES
WPB = NUM_WORKERS // SC_BATCHES
CHUNK = N // WPB

TC_CH = 256


def _round_bf16(x):
  t = x * jnp.float32(65537.0)
  return t - (t - x)


def _sc_body(tmpl_hbm, src_hbm, out_a_hbm, out_b_hbm,
             tmpl_v, src_v, trx_v, try_v, trz_v, srx_v, sry_v, srz_v,
             tnorm_v, snorm_v, min_a_v, min_b_v):
  c = lax.axis_index("c")
  s = lax.axis_index("s")
  wid = s * NUM_CORES + c
  b = wid // WPB
  base = (wid % WPB) * CHUNK

  pltpu.sync_copy(tmpl_hbm.at[b], tmpl_v)
  pltpu.sync_copy(src_hbm.at[b], src_v)

  def prep(f32_v, rx_v, ry_v, rz_v, norm_v, npts):
    def body(i, _):
      sl = pl.ds(i * LANES, LANES)
      x = f32_v[0, sl]
      y = f32_v[1, sl]
      z = f32_v[2, sl]
      rx_v[sl] = _round_bf16(x)
      ry_v[sl] = _round_bf16(y)
      rz_v[sl] = _round_bf16(z)
      norm_v[sl] = x * x + y * y + z * z
      return 0
    lax.fori_loop(0, npts // LANES, body, 0)

  prep(tmpl_v, trx_v, try_v, trz_v, tnorm_v, N)
  prep(src_v, srx_v, sry_v, srz_v, snorm_v, M)

  lane_iota = lax.iota(jnp.int32, LANES)

  def lane_gather(vec, idx):
    dnums = lax.GatherDimensionNumbers(
        offset_dims=(), collapsed_slice_dims=(0,), start_index_map=(0,))
    return lax.gather(vec, idx[:, None], dnums, (1,),
                      mode=lax.GatherScatterMode.PROMISE_IN_BOUNDS)

  rot_idx = {k: (lane_iota + k) & (LANES - 1) for k in (1, 2, 4, 8)}
  splat_idx = [jnp.full((LANES,), j, jnp.int32) for j in range(LANES)]

  def one_direction(q_xyz, qnorm_v, p_xyz, pnorm_v, npts, out_v):
    qx_v, qy_v, qz_v = q_xyz
    px_v, py_v, pz_v = p_xyz
    def q_group(g16, _):
      qsl = pl.ds(base + g16 * LANES, LANES)
      qx16 = qx_v[qsl]
      qy16 = qy_v[qsl]
      qz16 = qz_v[qsl]
      outvec = jnp.zeros((LANES,), jnp.float32)
      for h in range(LANES // G):
        qx, qy, qz = [], [], []
        for r in range(G):
          j = h * G + r
          qx.append(lane_gather(qx16, splat_idx[j]) * -2.0)
          qy.append(lane_gather(qy16, splat_idx[j]) * -2.0)
          qz.append(lane_gather(qz16, splat_idx[j]) * -2.0)

        UNROLL = 4

        def scan_points(mv, best):
          best = list(best)
          for u in range(UNROLL):
            sl = pl.ds((mv * UNROLL + u) * LANES, LANES)
            px = px_v[sl]
            py = py_v[sl]
            pz = pz_v[sl]
            pn = pnorm_v[sl]
            for r in range(G):
              cst = qx[r] * px + pn
              cst = qy[r] * py + cst
              cst = qz[r] * pz + cst
              best[r] = jnp.minimum(best[r], cst)
          return tuple(best)

        init = tuple(jnp.full((LANES,), jnp.inf, jnp.float32)
                     for _ in range(G))
        best = lax.fori_loop(0, npts // (LANES * UNROLL), scan_points, init)
        for r in range(G):
          t = best[r]
          for k in (8, 4, 2, 1):
            t = jnp.minimum(t, lane_gather(t, rot_idx[k]))
          outvec = jnp.where(lane_iota == (h * G + r), t, outvec)

      nq = qnorm_v[qsl]
      out_v[pl.ds(g16 * LANES, LANES)] = outvec + nq
      return 0

    lax.fori_loop(0, CHUNK // LANES, q_group, 0)

  one_direction((trx_v, try_v, trz_v), tnorm_v,
                (srx_v, sry_v, srz_v), snorm_v, M, min_a_v)
  one_direction((srx_v, sry_v, srz_v), snorm_v,
                (trx_v, try_v, trz_v), tnorm_v, N, min_b_v)

  pltpu.sync_copy(min_a_v, out_a_hbm.at[b, pl.ds(base, CHUNK)])
  pltpu.sync_copy(min_b_v, out_b_hbm.at[b, pl.ds(base, CHUNK)])


_sc_min_dists = functools.partial(
    pl.kernel,
    out_type=[
        jax.ShapeDtypeStruct((SC_BATCHES, N), jnp.float32),
        jax.ShapeDtypeStruct((SC_BATCHES, M), jnp.float32),
    ],
    mesh=plsc.VectorSubcoreMesh(
        core_axis_name="c", subcore_axis_name="s",
        num_cores=NUM_CORES, num_subcores=NUM_SUBCORES),
    scratch_types=[
        pltpu.VMEM((3, N), jnp.float32),
        pltpu.VMEM((3, M), jnp.float32),
        pltpu.VMEM((N,), jnp.float32),
        pltpu.VMEM((N,), jnp.float32),
        pltpu.VMEM((N,), jnp.float32),
        pltpu.VMEM((M,), jnp.float32),
        pltpu.VMEM((M,), jnp.float32),
        pltpu.VMEM((M,), jnp.float32),
        pltpu.VMEM((N,), jnp.float32),
        pltpu.VMEM((M,), jnp.float32),
        pltpu.VMEM((CHUNK,), jnp.float32),
        pltpu.VMEM((CHUNK,), jnp.float32),
    ],
)(_sc_body)


def _tc_body(ta_ref, t_ref, s_ref, out_a_ref, out_b_ref):
  tx = t_ref[0, 0, :]
  ty = t_ref[0, 1, :]
  tz = t_ref[0, 2, :]
  sx = s_ref[0, 0, :]
  sy = s_ref[0, 1, :]
  sz = s_ref[0, 2, :]

  rhs = s_ref[0].astype(jnp.bfloat16)
  sn = (sx * sx + sy * sy + sz * sz).reshape(1, M)
  tn = tx * tx + ty * ty + tz * tz

  colmin = jnp.full((1, M), jnp.inf, jnp.float32)
  for c in range(N // TC_CH):
    lo, hi = c * TC_CH, (c + 1) * TC_CH
    lhs = ta_ref[0, lo:hi, :].astype(jnp.bfloat16) * jnp.bfloat16(-2.0)
    inner = lax.dot_general(lhs, rhs, (((1,), (0,)), ((), ())),
                            preferred_element_type=jnp.float32)
    d = inner + (tn[lo:hi].reshape(TC_CH, 1) + sn)
    out_a_ref[0, 0, lo:hi] = jnp.min(d, axis=1)
    colmin = jnp.minimum(colmin, jnp.min(d, axis=0, keepdims=True))
  out_b_ref[0, 0, :] = colmin[0, :]


_tc_min_dists = pl.pallas_call(
    _tc_body,
    grid=(TC_BATCHES,),
    in_specs=[
        pl.BlockSpec((1, N, 3), lambda i: (i, 0, 0)),
        pl.BlockSpec((1, 3, N), lambda i: (i, 0, 0)),
        pl.BlockSpec((1, 3, M), lambda i: (i, 0, 0)),
    ],
    out_specs=[
        pl.BlockSpec((1, 1, N), lambda i: (i, 0, 0)),
        pl.BlockSpec((1, 1, M), lambda i: (i, 0, 0)),
    ],
    out_shape=[
        jax.ShapeDtypeStruct((TC_BATCHES, 1, N), jnp.float32),
        jax.ShapeDtypeStruct((TC_BATCHES, 1, M), jnp.float32),
    ],
)


def _finish_body(sa_ref, sb_ref, ta_ref, tb_ref, o_ref):
  acc = jnp.sum(jnp.sqrt(jnp.maximum(sa_ref[...], 1e-12)))
  acc += jnp.sum(jnp.sqrt(jnp.maximum(sb_ref[...], 1e-12)))
  acc += jnp.sum(jnp.sqrt(jnp.maximum(ta_ref[...], 1e-12)))
  acc += jnp.sum(jnp.sqrt(jnp.maximum(tb_ref[...], 1e-12)))
  o_ref[0, 0] = acc * (0.5 / (B * N))


_finish = pl.pallas_call(
    _finish_body,
    out_shape=jax.ShapeDtypeStruct((1, 1), jnp.float32),
    out_specs=pl.BlockSpec(memory_space=pltpu.SMEM),
)


@jax.jit
def kernel(template, source):
  tmpl_soa = jnp.transpose(template, (0, 2, 1))
  src_soa = jnp.transpose(source, (0, 2, 1))
  sc_a, sc_b = _sc_min_dists(tmpl_soa[:SC_BATCHES], src_soa[:SC_BATCHES])
  tc_a, tc_b = _tc_min_dists(template[SC_BATCHES:], tmpl_soa[SC_BATCHES:],
                             src_soa[SC_BATCHES:])
  return _finish(sc_a, sc_b, tc_a, tc_b)[0, 0]


# speedup vs baseline: 1.4090x; 1.4090x over previous
import functools

import jax
import jax.numpy as jnp
from jax import lax
from jax.experimental import pallas as pl
from jax.experimental.pallas import tpu as pltpu
from jax.experimental.pallas import tpu_sc as plsc

B = 16
N = 2048
M = 2048
LANES = 16
NUM_CORES = 2
NUM_SUBCORES = 16
NUM_WORKERS = NUM_CORES * NUM_SUBCORES
G = 8

SC_BATCHES = 2
TC_BATCHES = B - SC_BATCHES
WPB = NUM_WORKERS // SC_BATCHES
CHUNK = N // WPB

TC_CH = 512


def _round_bf16(x):
  t = x * jnp.float32(65537.0)
  return t - (t - x)


def _sc_body(tmpl_hbm, src_hbm, out_a_hbm, out_b_hbm,
             tmpl_v, src_v, trx_v, try_v, trz_v, srx_v, sry_v, srz_v,
             tnorm_v, snorm_v, min_a_v, min_b_v):
  c = lax.axis_index("c")
  s = lax.axis_index("s")
  wid = s * NUM_CORES + c
  b = wid // WPB
  base = (wid % WPB) * CHUNK

  pltpu.sync_copy(tmpl_hbm.at[b], tmpl_v)
  pltpu.sync_copy(src_hbm.at[b], src_v)

  def prep(f32_v, rx_v, ry_v, rz_v, norm_v, npts):
    def body(i, _):
      sl = pl.ds(i * LANES, LANES)
      x = f32_v[0, sl]
      y = f32_v[1, sl]
      z = f32_v[2, sl]
      rx_v[sl] = _round_bf16(x)
      ry_v[sl] = _round_bf16(y)
      rz_v[sl] = _round_bf16(z)
      norm_v[sl] = x * x + y * y + z * z
      return 0
    lax.fori_loop(0, npts // LANES, body, 0)

  prep(tmpl_v, trx_v, try_v, trz_v, tnorm_v, N)
  prep(src_v, srx_v, sry_v, srz_v, snorm_v, M)

  lane_iota = lax.iota(jnp.int32, LANES)

  def lane_gather(vec, idx):
    dnums = lax.GatherDimensionNumbers(
        offset_dims=(), collapsed_slice_dims=(0,), start_index_map=(0,))
    return lax.gather(vec, idx[:, None], dnums, (1,),
                      mode=lax.GatherScatterMode.PROMISE_IN_BOUNDS)

  rot_idx = {k: (lane_iota + k) & (LANES - 1) for k in (1, 2, 4, 8)}
  splat_idx = [jnp.full((LANES,), j, jnp.int32) for j in range(LANES)]

  def one_direction(q_xyz, qnorm_v, p_xyz, pnorm_v, npts, out_v):
    qx_v, qy_v, qz_v = q_xyz
    px_v, py_v, pz_v = p_xyz
    def q_group(g16, _):
      qsl = pl.ds(base + g16 * LANES, LANES)
      qx16 = qx_v[qsl]
      qy16 = qy_v[qsl]
      qz16 = qz_v[qsl]
      outvec = jnp.zeros((LANES,), jnp.float32)
      for h in range(LANES // G):
        qx, qy, qz = [], [], []
        for r in range(G):
          j = h * G + r
          qx.append(lane_gather(qx16, splat_idx[j]) * -2.0)
          qy.append(lane_gather(qy16, splat_idx[j]) * -2.0)
          qz.append(lane_gather(qz16, splat_idx[j]) * -2.0)

        def scan_points(mv, best):
          sl = pl.ds(mv * LANES, LANES)
          px = px_v[sl]
          py = py_v[sl]
          pz = pz_v[sl]
          pn = pnorm_v[sl]
          out = []
          for r in range(G):
            cst = qx[r] * px + pn
            cst = qy[r] * py + cst
            cst = qz[r] * pz + cst
            out.append(jnp.minimum(best[r], cst))
          return tuple(out)

        init = tuple(jnp.full((LANES,), jnp.inf, jnp.float32)
                     for _ in range(G))
        best = lax.fori_loop(0, npts // LANES, scan_points, init)
        for r in range(G):
          t = best[r]
          for k in (8, 4, 2, 1):
            t = jnp.minimum(t, lane_gather(t, rot_idx[k]))
          outvec = jnp.where(lane_iota == (h * G + r), t, outvec)

      nq = qnorm_v[qsl]
      out_v[pl.ds(g16 * LANES, LANES)] = outvec + nq
      return 0

    lax.fori_loop(0, CHUNK // LANES, q_group, 0)

  one_direction((trx_v, try_v, trz_v), tnorm_v,
                (srx_v, sry_v, srz_v), snorm_v, M, min_a_v)
  one_direction((srx_v, sry_v, srz_v), snorm_v,
                (trx_v, try_v, trz_v), tnorm_v, N, min_b_v)

  pltpu.sync_copy(min_a_v, out_a_hbm.at[b, pl.ds(base, CHUNK)])
  pltpu.sync_copy(min_b_v, out_b_hbm.at[b, pl.ds(base, CHUNK)])


_sc_min_dists = functools.partial(
    pl.kernel,
    out_type=[
        jax.ShapeDtypeStruct((SC_BATCHES, N), jnp.float32),
        jax.ShapeDtypeStruct((SC_BATCHES, M), jnp.float32),
    ],
    mesh=plsc.VectorSubcoreMesh(
        core_axis_name="c", subcore_axis_name="s",
        num_cores=NUM_CORES, num_subcores=NUM_SUBCORES),
    scratch_types=[
        pltpu.VMEM((3, N), jnp.float32),
        pltpu.VMEM((3, M), jnp.float32),
        pltpu.VMEM((N,), jnp.float32),
        pltpu.VMEM((N,), jnp.float32),
        pltpu.VMEM((N,), jnp.float32),
        pltpu.VMEM((M,), jnp.float32),
        pltpu.VMEM((M,), jnp.float32),
        pltpu.VMEM((M,), jnp.float32),
        pltpu.VMEM((N,), jnp.float32),
        pltpu.VMEM((M,), jnp.float32),
        pltpu.VMEM((CHUNK,), jnp.float32),
        pltpu.VMEM((CHUNK,), jnp.float32),
    ],
)(_sc_body)


def _tc_body(ta_ref, t_ref, s_ref, out_a_ref, out_b_ref):
  tx = t_ref[0, 0, :]
  ty = t_ref[0, 1, :]
  tz = t_ref[0, 2, :]
  sx = s_ref[0, 0, :]
  sy = s_ref[0, 1, :]
  sz = s_ref[0, 2, :]

  rhs = s_ref[0].astype(jnp.bfloat16)
  sn = (sx * sx + sy * sy + sz * sz).reshape(1, M)
  tn = tx * tx + ty * ty + tz * tz

  inners = []
  for c in range(N // TC_CH):
    lo, hi = c * TC_CH, (c + 1) * TC_CH
    lhs = ta_ref[0, lo:hi, :].astype(jnp.bfloat16) * jnp.bfloat16(-2.0)
    inners.append(lax.dot_general(lhs, rhs, (((1,), (0,)), ((), ())),
                                  preferred_element_type=jnp.float32))
  colmin = jnp.full((1, M), jnp.inf, jnp.float32)
  for c in range(N // TC_CH):
    lo, hi = c * TC_CH, (c + 1) * TC_CH
    d = inners[c] + (tn[lo:hi].reshape(TC_CH, 1) + sn)
    out_a_ref[0, 0, lo:hi] = jnp.min(d, axis=1)
    colmin = jnp.minimum(colmin, jnp.min(d, axis=0, keepdims=True))
  out_b_ref[0, 0, :] = colmin[0, :]


_tc_min_dists = pl.pallas_call(
    _tc_body,
    grid=(TC_BATCHES,),
    in_specs=[
        pl.BlockSpec((1, N, 3), lambda i: (i, 0, 0)),
        pl.BlockSpec((1, 3, N), lambda i: (i, 0, 0)),
        pl.BlockSpec((1, 3, M), lambda i: (i, 0, 0)),
    ],
    out_specs=[
        pl.BlockSpec((1, 1, N), lambda i: (i, 0, 0)),
        pl.BlockSpec((1, 1, M), lambda i: (i, 0, 0)),
    ],
    out_shape=[
        jax.ShapeDtypeStruct((TC_BATCHES, 1, N), jnp.float32),
        jax.ShapeDtypeStruct((TC_BATCHES, 1, M), jnp.float32),
    ],
)


def _finish_body(sa_ref, sb_ref, ta_ref, tb_ref, o_ref):
  acc = jnp.sum(jnp.sqrt(jnp.maximum(sa_ref[...], 1e-12)))
  acc += jnp.sum(jnp.sqrt(jnp.maximum(sb_ref[...], 1e-12)))
  acc += jnp.sum(jnp.sqrt(jnp.maximum(ta_ref[...], 1e-12)))
  acc += jnp.sum(jnp.sqrt(jnp.maximum(tb_ref[...], 1e-12)))
  o_ref[0, 0] = acc * (0.5 / (B * N))


_finish = pl.pallas_call(
    _finish_body,
    out_shape=jax.ShapeDtypeStruct((1, 1), jnp.float32),
    out_specs=pl.BlockSpec(memory_space=pltpu.SMEM),
)


@jax.jit
def kernel(template, source):
  tmpl_soa = jnp.transpose(template, (0, 2, 1))
  src_soa = jnp.transpose(source, (0, 2, 1))
  sc_a, sc_b = _sc_min_dists(tmpl_soa[:SC_BATCHES], src_soa[:SC_BATCHES])
  tc_a, tc_b = _tc_min_dists(template[SC_BATCHES:], tmpl_soa[SC_BATCHES:],
                             src_soa[SC_BATCHES:])
  return _finish(sc_a, sc_b, tc_a, tc_b)[0, 0]
